# baseline (speedup 1.0000x reference)
import dataclasses
import functools

import jax
import jax.numpy as jnp
from jax import lax
from jax.experimental import pallas as pl
from jax.experimental.pallas import tpu as pltpu
from jax.experimental.pallas import tpu_sc as plsc

MAX_LETTERS = 20
HIDDEN = 128
PACKED = HIDDEN
LANES = 16

NUM_CORES = 2
NUM_SUBCORES = 16
NUM_WORKERS = NUM_CORES * NUM_SUBCORES

CHUNK_W = 20
CHUNK_I = CHUNK_W * MAX_LETTERS
PAIR_I = 2 * CHUNK_I
SUB = 80
NSUB = CHUNK_I // SUB


def _sc_embed(ids_pairs, tri_pack, pos_pack, total_chunks):
    chunks_per_worker = total_chunks // NUM_WORKERS
    pairs = chunks_per_worker // 2
    mesh = plsc.VectorSubcoreMesh(core_axis_name="c", subcore_axis_name="s")
    cp = pltpu.CompilerParams()
    if "needs_layout_passes" in pltpu.CompilerParams.__dataclass_fields__:
        cp = dataclasses.replace(cp, needs_layout_passes=False)

    @functools.partial(
        pl.kernel,
        mesh=mesh,
        compiler_params=cp,
        out_type=jax.ShapeDtypeStruct((total_chunks, CHUNK_W, PACKED),
                                      jnp.float32),
        scratch_types=[
            pltpu.VMEM((2 * NSUB, SUB), jnp.int32),
            pltpu.VMEM((2 * NSUB, SUB), jnp.int32),
            pltpu.VMEM((2, CHUNK_I, PACKED), jnp.float32),
            pltpu.VMEM((2, CHUNK_W, PACKED), jnp.float32),
            pltpu.VMEM((CHUNK_W, PACKED), jnp.float32),
            pltpu.SemaphoreType.DMA,
            pltpu.SemaphoreType.DMA,
            pltpu.SemaphoreType.DMA,
            pltpu.SemaphoreType.DMA,
        ],
    )
    def k(ids_hbm, tri_hbm, pos_hbm, out_hbm,
          idxp0, idxp1, rows_v, obuf_v, pos_v, semg0, semg1, semo0, semo1):
        idx_v = (idxp0, idxp1)
        cid = lax.axis_index("c")
        sid = lax.axis_index("s")
        wid = sid * NUM_CORES + cid
        chunk_base = wid * chunks_per_worker
        pair_base = wid * pairs
        semg = (semg0, semg1)
        semo = (semo0, semo1)

        def start_gather(pp, b):
            for s in range(NSUB):
                pltpu.async_copy(
                    tri_hbm.at[idx_v[pp].at[b * NSUB + s]],
                    rows_v.at[b].at[pl.ds(s * SUB, SUB)], semg[b])

        def wait_gather(pp, b):
            for s in range(NSUB):
                pltpu.make_async_copy(
                    tri_hbm.at[idx_v[pp].at[b * NSUB + s]],
                    rows_v.at[b].at[pl.ds(s * SUB, SUB)], semg[b]).wait()

        def reduce_chunk(b):
            rows = rows_v.at[b]
            obuf = obuf_v.at[b]

            @pl.loop(0, CHUNK_W)
            def _(w):
                rbase = w * MAX_LETTERS
                for c in range(PACKED // LANES):
                    sl = pl.ds(c * LANES, LANES)
                    acc = pos_v[w, sl]
                    for l in range(MAX_LETTERS):
                        acc = acc + rows[rbase + l, sl]
                    obuf[w, sl] = acc

        pltpu.sync_copy(pos_hbm, pos_v)
        pltpu.sync_copy(ids_hbm.at[pair_base], idx_v[0])
        start_gather(0, 0)

        def do_pair(j, pp):
            @pl.when(j + 1 < pairs)
            def _():
                pltpu.sync_copy(ids_hbm.at[pair_base + j + 1], idx_v[pp ^ 1])
            wait_gather(pp, 0)
            start_gather(pp, 1)
            @pl.when(j >= 1)
            def _():
                pltpu.make_async_copy(
                    obuf_v.at[0], out_hbm.at[chunk_base + 2 * (j - 1)],
                    semo[0]).wait()
            reduce_chunk(0)
            pltpu.async_copy(obuf_v.at[0], out_hbm.at[chunk_base + 2 * j],
                             semo[0])
            wait_gather(pp, 1)
            @pl.when(j + 1 < pairs)
            def _():
                start_gather(pp ^ 1, 0)
            @pl.when(j >= 1)
            def _():
                pltpu.make_async_copy(
                    obuf_v.at[1], out_hbm.at[chunk_base + 2 * (j - 1) + 1],
                    semo[1]).wait()
            reduce_chunk(1)
            pltpu.async_copy(obuf_v.at[1], out_hbm.at[chunk_base + 2 * j + 1],
                             semo[1])

        @pl.loop(0, pairs, step=2)
        def _(jj):
            do_pair(jj, 0)
            do_pair(jj + 1, 1)

        pltpu.make_async_copy(
            obuf_v.at[0], out_hbm.at[chunk_base + 2 * (pairs - 1)],
            semo[0]).wait()
        pltpu.make_async_copy(
            obuf_v.at[1], out_hbm.at[chunk_base + 2 * (pairs - 1) + 1],
            semo[1]).wait()

    return k(ids_pairs, tri_pack, pos_pack)


def kernel(input_ids, tri_table, pos_table):
    batch = input_ids.shape[0]
    seq_len = input_ids.shape[1] // MAX_LETTERS
    total_chunks = batch * seq_len // CHUNK_W

    ids_pairs = input_ids.reshape(-1, 2 * NSUB, SUB)
    pos_rows = pos_table[1:seq_len + 1]

    out = _sc_embed(ids_pairs, tri_table, pos_rows, total_chunks)
    return out.reshape(batch, seq_len, HIDDEN)


# speedup vs baseline: 1.0926x; 1.0926x over previous
import functools

import jax
import jax.numpy as jnp
from jax import lax
from jax.experimental import pallas as pl
from jax.experimental.pallas import tpu as pltpu
from jax.experimental.pallas import tpu_sc as plsc

MAX_LETTERS = 20
HIDDEN = 128

NUM_CORES = 2
NUM_SUBCORES = 16
NUM_WORKERS = NUM_CORES * NUM_SUBCORES

CHUNK_W = 20
CHUNK_I = CHUNK_W * MAX_LETTERS
PAIR_W = 2 * CHUNK_W
PAIR_I = 2 * CHUNK_I
SUB = 100
NSUB = CHUNK_I // SUB
REGION = 2 * PAIR_W


def _sc_embed(ids_pairs, tri_table, pos_rep, widx_pat, total_words):
    words_per_worker = total_words // NUM_WORKERS
    pairs = words_per_worker // PAIR_W
    mesh = plsc.VectorSubcoreMesh(core_axis_name="c", subcore_axis_name="s")

    @functools.partial(
        pl.kernel,
        mesh=mesh,
        out_type=jax.ShapeDtypeStruct((total_words, HIDDEN), jnp.float32),
        scratch_types=[
            pltpu.VMEM((2 * NSUB, SUB), jnp.int32),
            pltpu.VMEM((2 * NSUB, SUB), jnp.int32),
            pltpu.VMEM((2, CHUNK_I, HIDDEN), jnp.float32),
            pltpu.VMEM_SHARED((NUM_SUBCORES * REGION, HIDDEN), jnp.float32),
            pltpu.VMEM((NSUB, SUB), jnp.int32),
            pltpu.VMEM((NSUB, SUB), jnp.int32),
            pltpu.VMEM((NSUB, SUB), jnp.int32),
            pltpu.VMEM((NSUB, SUB), jnp.int32),
            pltpu.SemaphoreType.DMA,
            pltpu.SemaphoreType.DMA,
            pltpu.SemaphoreType.DMA,
        ],
    )
    def k(ids_hbm, tri_hbm, pos_hbm, widx_hbm, out_hbm,
          idxp0, idxp1, rows_v, accum_sh, w00, w01, w10, w11,
          semg0, semg1, semo):
        idx_v = (idxp0, idxp1)
        widx = ((w00, w01), (w10, w11))
        cid = lax.axis_index("c")
        sid = lax.axis_index("s")
        wid = sid * NUM_CORES + cid
        word_base = wid * words_per_worker
        pair_base = wid * pairs
        abase = sid * REGION
        region = (accum_sh.at[pl.ds(abase, PAIR_W)],
                  accum_sh.at[pl.ds(abase + PAIR_W, PAIR_W)])
        semg = (semg0, semg1)

        def start_gather(pp, b):
            for s in range(NSUB):
                pltpu.async_copy(
                    tri_hbm.at[idx_v[pp].at[b * NSUB + s]],
                    rows_v.at[b].at[pl.ds(s * SUB, SUB)], semg[b])

        def wait_gather(pp, b):
            for s in range(NSUB):
                pltpu.make_async_copy(
                    tri_hbm.at[idx_v[pp].at[b * NSUB + s]],
                    rows_v.at[b].at[pl.ds(s * SUB, SUB)], semg[b]).wait()

        def scatter(pp, b):
            for s in range(NSUB):
                pltpu.sync_copy(rows_v.at[b].at[pl.ds(s * SUB, SUB)],
                                accum_sh.at[widx[pp][b].at[s]], add=True)

        for pp in range(2):
            for b in range(2):
                pltpu.sync_copy(widx_hbm.at[sid].at[pp].at[b], widx[pp][b])
        pltpu.sync_copy(ids_hbm.at[pair_base], idx_v[0])
        start_gather(0, 0)

        def do_pair(j, pp):
            @pl.when(j + 1 < pairs)
            def _():
                pltpu.sync_copy(ids_hbm.at[pair_base + j + 1], idx_v[pp ^ 1])
            wait_gather(pp, 0)
            start_gather(pp, 1)
            @pl.when(j >= 2)
            def _():
                pltpu.make_async_copy(
                    region[pp],
                    out_hbm.at[pl.ds(word_base + (j - 2) * PAIR_W, PAIR_W)],
                    semo).wait()
            pltpu.sync_copy(pos_hbm, region[pp])
            scatter(pp, 0)
            @pl.when(j >= 1)
            def _():
                pltpu.async_copy(
                    region[pp ^ 1],
                    out_hbm.at[pl.ds(word_base + (j - 1) * PAIR_W, PAIR_W)],
                    semo)
            wait_gather(pp, 1)
            @pl.when(j + 1 < pairs)
            def _():
                start_gather(pp ^ 1, 0)
            scatter(pp, 1)

        @pl.loop(0, pairs, step=2)
        def _(jj):
            do_pair(jj, 0)
            do_pair(jj + 1, 1)

        pltpu.make_async_copy(
            region[0],
            out_hbm.at[pl.ds(word_base + (pairs - 2) * PAIR_W, PAIR_W)],
            semo).wait()
        pltpu.sync_copy(pos_hbm, region[0])
        pltpu.sync_copy(pos_hbm, region[0])
        pltpu.sync_copy(
            region[1],
            out_hbm.at[pl.ds(word_base + (pairs - 1) * PAIR_W, PAIR_W)])

    return k(ids_pairs, tri_table, pos_rep, widx_pat)


def kernel(input_ids, tri_table, pos_table):
    batch = input_ids.shape[0]
    seq_len = input_ids.shape[1] // MAX_LETTERS
    total_words = batch * seq_len

    ids_pairs = input_ids.reshape(-1, 2 * NSUB, SUB)
    pos_rep = jnp.tile(pos_table[1:seq_len + 1], (PAIR_W // seq_len, 1))
    base = jnp.arange(CHUNK_I, dtype=jnp.int32) // MAX_LETTERS
    widx_pat = (jnp.arange(NUM_SUBCORES, dtype=jnp.int32)[:, None, None, None]
                * REGION
                + jnp.arange(2, dtype=jnp.int32)[None, :, None, None] * PAIR_W
                + jnp.arange(2, dtype=jnp.int32)[None, None, :, None] * CHUNK_W
                + base.reshape(1, 1, 1, CHUNK_I)
                ).reshape(NUM_SUBCORES, 2, 2, NSUB, SUB)

    out = _sc_embed(ids_pairs, tri_table, pos_rep, widx_pat, total_words)
    return out.reshape(batch, seq_len, HIDDEN)
